# Initial kernel scaffold; baseline (speedup 1.0000x reference)
import jax
import jax.numpy as jnp
from jax.experimental import pallas as pl


def kernel(x_low, x_high, attention):
    raise NotImplementedError("write your pallas kernel here")


# speedup vs baseline: 2.9926x; 2.9926x over previous
import functools

import jax
import jax.numpy as jnp
from jax import lax
from jax.experimental import pallas as pl
from jax.experimental.pallas import tpu as pltpu
from jax.experimental.pallas import tpu_sc as plsc

N_PATCHES = 32
PATCH = 64
ATT_W = 64
IMG_H = 1024
IMG_W3 = 3 * 1024
ROW_W3 = 3 * PATCH


def _topk_coords_body(scores_ref, flat_ref, att_ref, r_ref, c3_ref):
    scores = scores_ref[...]
    flat = flat_ref[...]
    col = lax.broadcasted_iota(jnp.int32, scores.shape, 1)
    idx_list = []
    val_list = []
    for _ in range(N_PATCHES):
        m = jnp.max(scores, axis=1, keepdims=True)
        is_max = scores == m
        idx = jnp.min(jnp.where(is_max, col, scores.shape[1]), axis=1)
        sel = col == idx[:, None]
        val = jnp.sum(jnp.where(sel, flat, 0.0), axis=1)
        scores = jnp.where(sel, -jnp.inf, scores)
        idx_list.append(idx)
        val_list.append(val)
    idxs = jnp.stack(idx_list, axis=1)
    att_ref[...] = jnp.stack(val_list, axis=1)

    rows = idxs // ATT_W
    cols = idxs % ATT_W
    def to_tl(s):
        c_high = s.astype(jnp.float32) / 63.0 * 255.0 * 4.0
        tl = jnp.round(c_high - float(PATCH) / 2.0).astype(jnp.int32)
        return jnp.clip(tl, 0, IMG_H - PATCH)

    r_ref[...] = to_tl(rows)
    c3_ref[...] = to_tl(cols) * 3


def _topk_coords(scores, flat):
    B = scores.shape[0]
    return pl.pallas_call(
        _topk_coords_body,
        out_shape=[
            jax.ShapeDtypeStruct((B, N_PATCHES), jnp.float32),
            jax.ShapeDtypeStruct((B, N_PATCHES), jnp.int32),
            jax.ShapeDtypeStruct((B, N_PATCHES), jnp.int32),
        ],
    )(scores, flat)


def _make_gather(B):
    n_workers = 32
    pairs = B * N_PATCHES
    per_w = pairs // n_workers
    mesh = plsc.VectorSubcoreMesh(core_axis_name="c", subcore_axis_name="s")

    W_AL = ROW_W3 + 8
    CS_MAX = IMG_W3 - W_AL

    @functools.partial(
        pl.kernel,
        out_type=jax.ShapeDtypeStruct((B, N_PATCHES, PATCH, ROW_W3), jnp.float32),
        mesh=mesh,
        compiler_params=pltpu.CompilerParams(use_tc_tiling_on_sc=False),
        scratch_types=[
            pltpu.VMEM((16,), jnp.int32),
            pltpu.VMEM((16,), jnp.int32),
            pltpu.VMEM((2, PATCH, W_AL), jnp.float32),
            pltpu.VMEM((2, PATCH, ROW_W3), jnp.float32),
            pltpu.SemaphoreType.DMA((2,)),
            pltpu.SemaphoreType.DMA((2,)),
        ],
    )
    def gather(xh_hbm, r_hbm, c3_hbm, out_hbm, r_v, c3_v, buf, outb, in_sem, out_sem):
        wid = lax.axis_index("s") * 2 + lax.axis_index("c")
        b = wid // (N_PATCHES // per_w)
        n0 = (wid % (N_PATCHES // per_w)) * per_w
        pltpu.sync_copy(r_hbm.at[b, pl.ds(n0, per_w)], r_v.at[pl.ds(0, per_w)])
        pltpu.sync_copy(c3_hbm.at[b, pl.ds(n0, per_w)], c3_v.at[pl.ds(0, per_w)])
        r_vec = r_v[...]
        c3_vec = c3_v[...]

        def dma_in(j, slot):
            r = r_vec[j]
            c3 = c3_vec[j]
            cs = jnp.minimum((c3 // 8) * 8, CS_MAX)
            return pltpu.make_async_copy(
                xh_hbm.at[b, pl.ds(r, PATCH), pl.ds(cs, W_AL)],
                buf.at[slot],
                in_sem.at[slot],
            )

        def dma_out(j, slot):
            return pltpu.make_async_copy(
                outb.at[slot],
                out_hbm.at[b, n0 + j],
                out_sem.at[slot],
            )

        dma_in(0, 0).start()
        for j in range(per_w):
            slot = j % 2
            c3 = c3_vec[j]
            off = c3 - jnp.minimum((c3 // 8) * 8, CS_MAX)
            dma_in(j, slot).wait()
            if j + 1 < per_w:
                dma_in(j + 1, 1 - slot).start()
            if j >= 2:
                dma_out(j - 2, slot).wait()

            def shift_row(i, off):
                for m in range(ROW_W3 // 16):
                    outb[slot, i, pl.ds(16 * m, 16)] = buf[slot, i, pl.ds(off + 16 * m, 16)]
                return off

            lax.fori_loop(0, PATCH, shift_row, off)
            dma_out(j, slot).start()
        dma_out(per_w - 2, 0 if per_w % 2 == 0 else 1).wait()
        dma_out(per_w - 1, 1 if per_w % 2 == 0 else 0).wait()

    return gather


def kernel(x_low, x_high, attention):
    B = attention.shape[0]
    flat = attention.reshape(B, -1)
    logits = jnp.log(flat)
    key = jax.random.key(42)
    u = jax.random.uniform(key, logits.shape, minval=1e-8, maxval=1.0)
    z = -jnp.log(-jnp.log(u))
    scores = logits + z

    att_vals, tl_r, tl_c3 = _topk_coords(scores, flat)

    xh = x_high.reshape(B, IMG_H, IMG_W3)
    patches_flat = _make_gather(B)(xh, tl_r, tl_c3)
    patches = patches_flat.reshape(B, N_PATCHES, PATCH, PATCH, 3)
    return patches, att_vals
